# Initial kernel scaffold; baseline (speedup 1.0000x reference)
import jax
import jax.numpy as jnp
from jax.experimental import pallas as pl


def kernel(z_user, z_item, edge_index):
    raise NotImplementedError("write your pallas kernel here")


# speedup vs baseline: 1.0964x; 1.0964x over previous
import functools

import jax
import jax.numpy as jnp
from jax import lax
from jax.experimental import pallas as pl
from jax.experimental.pallas import tpu as pltpu
from jax.experimental.pallas import tpu_sc as plsc

E = 320000
D = 128
NC = 2
NS = 16
NW = NC * NS
EPW = E // NW
C = 80
NCHUNK = EPW // C
G = C // 16


def _sc_body(zu_hbm, zi_hbm, src_hbm, dst_hbm, out_hbm,
             sidx_v, didx_v, srow_v, drow_v, out_v, sem):
    wid = lax.axis_index("s") * NC + lax.axis_index("c")
    base = wid * EPW

    @pl.loop(0, NCHUNK)
    def chunk_loop(ci):
        cbase = base + ci * C
        pltpu.sync_copy(src_hbm.at[pl.ds(cbase, C)], sidx_v)
        pltpu.sync_copy(dst_hbm.at[pl.ds(cbase, C)], didx_v)
        cp_s = pltpu.async_copy(zu_hbm.at[sidx_v], srow_v, sem)
        cp_d = pltpu.async_copy(zi_hbm.at[didx_v], drow_v, sem)
        cp_s.wait()
        cp_d.wait()
        for g in range(G):
            lanes = lax.iota(jnp.int32, 16) + (g * 16)
            acc0 = jnp.zeros((16,), jnp.float32)
            dv0 = jnp.zeros((16,), jnp.int32)

            @pl.loop(0, D, init_carry=(acc0, dv0), unroll=8)
            def dot_loop(d, carry):
                acc, dv = carry
                s = plsc.load_gather(srow_v, [lanes, dv])
                t = plsc.load_gather(drow_v, [lanes, dv])
                return acc + s * t, dv + 1

            acc, _ = dot_loop
            sig = 1.0 / (1.0 + jnp.exp(-acc))
            out_v[pl.ds(ci * C + g * 16, 16)] = sig

    pltpu.sync_copy(out_v, out_hbm.at[pl.ds(base, EPW)])


@functools.partial(jax.jit, static_argnames=())
def _edge_decoder(z_user, z_item, src_idx, dst_idx):
    mesh = plsc.VectorSubcoreMesh(
        core_axis_name="c", subcore_axis_name="s",
        num_cores=NC, num_subcores=NS)
    return pl.kernel(
        _sc_body,
        out_type=jax.ShapeDtypeStruct((E,), jnp.float32),
        mesh=mesh,
        compiler_params=pltpu.CompilerParams(needs_layout_passes=False),
        scratch_types=[
            pltpu.VMEM((C,), jnp.int32),
            pltpu.VMEM((C,), jnp.int32),
            pltpu.VMEM((C, D), jnp.float32),
            pltpu.VMEM((C, D), jnp.float32),
            pltpu.VMEM((EPW,), jnp.float32),
            pltpu.SemaphoreType.DMA,
        ],
    )(z_user, z_item, src_idx, dst_idx)


def kernel(z_user, z_item, edge_index):
    src_idx = edge_index[0].astype(jnp.int32)
    dst_idx = edge_index[1].astype(jnp.int32)
    return _edge_decoder(z_user, z_item, src_idx, dst_idx)


# speedup vs baseline: 1.3442x; 1.2261x over previous
import functools

import jax
import jax.numpy as jnp
from jax import lax
from jax.experimental import pallas as pl
from jax.experimental.pallas import tpu as pltpu
from jax.experimental.pallas import tpu_sc as plsc

E = 320000
D = 128
NC = 2
NS = 16
NW = NC * NS
EPW = E // NW
C = 80
NCHUNK = EPW // C
G = C // 16
NBUF = 2


def _sc_body(zu_hbm, zi_hbm, src_hbm, dst_hbm, out_hbm,
             sidx0, sidx1, didx0, didx1, srow0, srow1, drow0, drow1,
             out_v, semr0, semr1, semi0, semi1):
    sidx = (sidx0, sidx1)
    didx = (didx0, didx1)
    srow = (srow0, srow1)
    drow = (drow0, drow1)
    semr = (semr0, semr1)
    semi = (semi0, semi1)

    wid = lax.axis_index("s") * NC + lax.axis_index("c")
    base = wid * EPW

    def fire_idx(ci, b):
        cbase = base + ci * C
        pltpu.async_copy(src_hbm.at[pl.ds(cbase, C)], sidx[b], semi[b])
        pltpu.async_copy(dst_hbm.at[pl.ds(cbase, C)], didx[b], semi[b])

    def drain_idx(b):
        pltpu.make_async_copy(src_hbm.at[pl.ds(0, C)], sidx[b], semi[b]).wait()
        pltpu.make_async_copy(dst_hbm.at[pl.ds(0, C)], didx[b], semi[b]).wait()

    def fire_rows(b):
        pltpu.async_copy(zu_hbm.at[sidx[b]], srow[b], semr[b])
        pltpu.async_copy(zi_hbm.at[didx[b]], drow[b], semr[b])

    def drain_rows(b):
        pltpu.make_async_copy(zu_hbm.at[sidx[b]], srow[b], semr[b]).wait()
        pltpu.make_async_copy(zi_hbm.at[didx[b]], drow[b], semr[b]).wait()

    def compute(ci, b):
        for g in range(G):
            lanes = lax.iota(jnp.int32, 16) + (g * 16)
            acc0 = jnp.zeros((16,), jnp.float32)
            dv0 = jnp.zeros((16,), jnp.int32)

            @pl.loop(0, D, init_carry=(acc0, dv0), unroll=8)
            def dot_loop(d, carry):
                acc, dv = carry
                s = plsc.load_gather(srow[b], [lanes, dv])
                t = plsc.load_gather(drow[b], [lanes, dv])
                return acc + s * t, dv + 1

            acc, _ = dot_loop
            sig = 1.0 / (1.0 + jnp.exp(-acc))
            out_v[pl.ds(ci * C + g * 16, 16)] = sig

    fire_idx(0, 0)
    drain_idx(0)
    fire_rows(0)
    fire_idx(1, 1)

    @pl.loop(0, NCHUNK - 1, step=NBUF)
    def chunk_loop(ci0):
        for b in range(NBUF):
            ci = ci0 + b
            bn = 1 - b

            @pl.when(ci + 1 < NCHUNK)
            def _():
                drain_idx(bn)
                fire_rows(bn)

            drain_rows(b)

            @pl.when(ci + NBUF < NCHUNK)
            def _():
                fire_idx(ci + NBUF, b)

            compute(ci, b)

    drain_rows(0)
    compute(NCHUNK - 1, 0)

    pltpu.sync_copy(out_v, out_hbm.at[pl.ds(base, EPW)])


@jax.jit
def _edge_decoder(z_user, z_item, src_idx, dst_idx):
    mesh = plsc.VectorSubcoreMesh(
        core_axis_name="c", subcore_axis_name="s",
        num_cores=NC, num_subcores=NS)
    return pl.kernel(
        _sc_body,
        out_type=jax.ShapeDtypeStruct((E,), jnp.float32),
        mesh=mesh,
        compiler_params=pltpu.CompilerParams(needs_layout_passes=False),
        scratch_types=[
            pltpu.VMEM((C,), jnp.int32),
            pltpu.VMEM((C,), jnp.int32),
            pltpu.VMEM((C,), jnp.int32),
            pltpu.VMEM((C,), jnp.int32),
            pltpu.VMEM((C, D), jnp.float32),
            pltpu.VMEM((C, D), jnp.float32),
            pltpu.VMEM((C, D), jnp.float32),
            pltpu.VMEM((C, D), jnp.float32),
            pltpu.VMEM((EPW,), jnp.float32),
            pltpu.SemaphoreType.DMA,
            pltpu.SemaphoreType.DMA,
            pltpu.SemaphoreType.DMA,
            pltpu.SemaphoreType.DMA,
        ],
    )(z_user, z_item, src_idx, dst_idx)


def kernel(z_user, z_item, edge_index):
    src_idx = edge_index[0].astype(jnp.int32)
    dst_idx = edge_index[1].astype(jnp.int32)
    return _edge_decoder(z_user, z_item, src_idx, dst_idx)


# speedup vs baseline: 2.3453x; 1.7447x over previous
import jax
import jax.numpy as jnp
from jax import lax
from jax.experimental import pallas as pl
from jax.experimental.pallas import tpu as pltpu
from jax.experimental.pallas import tpu_sc as plsc

E = 320000
D = 128
PW = D // 2
N = 10000
NC = 2
NS = 16
NW = NC * NS
EPW = E // NW
C = 80
NCHUNK = EPW // C
G = C // 16
NBUF = 2


def _sc_body(zu_hbm, zi_hbm, src_hbm, dst_hbm, out_hbm,
             sidx0, sidx1, didx0, didx1, srow0, srow1, drow0, drow1,
             out_v, semr0, semr1, semi0, semi1):
    sidx = (sidx0, sidx1)
    didx = (didx0, didx1)
    srow = (srow0, srow1)
    drow = (drow0, drow1)
    semr = (semr0, semr1)
    semi = (semi0, semi1)

    wid = lax.axis_index("s") * NC + lax.axis_index("c")
    base = wid * EPW

    def fire_idx(ci, b):
        cbase = base + ci * C
        pltpu.async_copy(src_hbm.at[pl.ds(cbase, C)], sidx[b], semi[b])
        pltpu.async_copy(dst_hbm.at[pl.ds(cbase, C)], didx[b], semi[b])

    def drain_idx(b):
        pltpu.make_async_copy(src_hbm.at[pl.ds(0, C)], sidx[b], semi[b]).wait()
        pltpu.make_async_copy(dst_hbm.at[pl.ds(0, C)], didx[b], semi[b]).wait()

    def fire_rows(b):
        pltpu.async_copy(zu_hbm.at[sidx[b]], srow[b], semr[b])
        pltpu.async_copy(zi_hbm.at[didx[b]], drow[b], semr[b])

    def drain_rows(b):
        pltpu.make_async_copy(zu_hbm.at[sidx[b]], srow[b], semr[b]).wait()
        pltpu.make_async_copy(zi_hbm.at[didx[b]], drow[b], semr[b]).wait()

    def compute(ci, b):
        for g in range(G):
            lanes = lax.iota(jnp.int32, 16) + (g * 16)
            acca0 = jnp.zeros((16,), jnp.float32)
            accb0 = jnp.zeros((16,), jnp.float32)
            dv0 = jnp.zeros((16,), jnp.int32)

            @pl.loop(0, PW, init_carry=(acca0, accb0, dv0), unroll=8)
            def dot_loop(d, carry):
                acca, accb, dv = carry
                si = plsc.load_gather(srow[b], [lanes, dv])
                ti = plsc.load_gather(drow[b], [lanes, dv])
                sbf = plsc.bitcast(si, jnp.bfloat16)
                tbf = plsc.bitcast(ti, jnp.bfloat16)
                q0, q1 = plsc.unpack(sbf * tbf,
                                     format=plsc.PackFormat.INTERLEAVED)
                return acca + q0, accb + q1, dv + 1

            acca, accb, _ = dot_loop
            acc = acca + accb
            sig = 1.0 / (1.0 + jnp.exp(-acc))
            out_v[pl.ds(ci * C + g * 16, 16)] = sig

    fire_idx(0, 0)
    drain_idx(0)
    fire_rows(0)
    fire_idx(1, 1)

    @pl.loop(0, NCHUNK - 1, step=NBUF)
    def chunk_loop(ci0):
        for b in range(NBUF):
            ci = ci0 + b
            bn = 1 - b

            @pl.when(ci + 1 < NCHUNK)
            def _():
                drain_idx(bn)
                fire_rows(bn)

            drain_rows(b)

            @pl.when(ci + NBUF < NCHUNK)
            def _():
                fire_idx(ci + NBUF, b)

            compute(ci, b)

    drain_rows(0)
    compute(NCHUNK - 1, 0)

    pltpu.sync_copy(out_v, out_hbm.at[pl.ds(base, EPW)])


@jax.jit
def _edge_decoder(zu_pk, zi_pk, src_idx, dst_idx):
    mesh = plsc.VectorSubcoreMesh(
        core_axis_name="c", subcore_axis_name="s",
        num_cores=NC, num_subcores=NS)
    return pl.kernel(
        _sc_body,
        out_type=jax.ShapeDtypeStruct((E,), jnp.float32),
        mesh=mesh,
        compiler_params=pltpu.CompilerParams(
            needs_layout_passes=False, use_tc_tiling_on_sc=False),
        scratch_types=[
            pltpu.VMEM((C,), jnp.int32),
            pltpu.VMEM((C,), jnp.int32),
            pltpu.VMEM((C,), jnp.int32),
            pltpu.VMEM((C,), jnp.int32),
            pltpu.VMEM((C, PW), jnp.int32),
            pltpu.VMEM((C, PW), jnp.int32),
            pltpu.VMEM((C, PW), jnp.int32),
            pltpu.VMEM((C, PW), jnp.int32),
            pltpu.VMEM((EPW,), jnp.float32),
            pltpu.SemaphoreType.DMA,
            pltpu.SemaphoreType.DMA,
            pltpu.SemaphoreType.DMA,
            pltpu.SemaphoreType.DMA,
        ],
    )(zu_pk, zi_pk, src_idx, dst_idx)


def kernel(z_user, z_item, edge_index):
    zu_pk = lax.bitcast_convert_type(
        z_user.astype(jnp.bfloat16).reshape(N, PW, 2), jnp.int32)
    zi_pk = lax.bitcast_convert_type(
        z_item.astype(jnp.bfloat16).reshape(N, PW, 2), jnp.int32)
    src_idx = edge_index[0].astype(jnp.int32)
    dst_idx = edge_index[1].astype(jnp.int32)
    return _edge_decoder(zu_pk, zi_pk, src_idx, dst_idx)
